# baseline (speedup 1.0000x reference)
import jax
import jax.numpy as jnp
from jax import lax
from jax.experimental import pallas as pl
from jax.experimental.pallas import tpu as pltpu
from jax.experimental.pallas import tpu_sc as plsc

_info = plsc.get_sparse_core_info()
_NC, _NS, _L = _info.num_cores, _info.num_subcores, _info.num_lanes
_NW = _NC * _NS

_B = 16384
_D = 64
_BPW = _B // _NW
_CH = 128
_NCHUNK = _BPW // _CH
_NBUF = 2
_PITCH = 137


def _film_body(xt_hbm, idx_hbm, film_hbm, out_hbm,
               idx_v, xt_v, rows0, rows1,
               sem_x, sem_g0, sem_g1, sem_st):
    wid = lax.axis_index("s") * _NC + lax.axis_index("c")
    base = wid * _BPW

    pltpu.sync_copy(idx_hbm.at[pl.ds(base, _BPW)], idx_v)

    rows = (rows0, rows1)
    sems = (sem_g0, sem_g1)
    gd = [None] * _NCHUNK
    for c in range(_NBUF):
        gd[c] = pltpu.async_copy(
            film_hbm.at[idx_v.at[pl.ds(c * _CH, _CH)]],
            rows[c].at[:, pl.ds(0, 2 * _D)], sems[c])
    xd = [pltpu.async_copy(xt_hbm.at[:, pl.ds(base + c * _CH, _CH)],
                           xt_v.at[:, pl.ds(c * _CH, _CH)], sem_x)
          for c in range(_NCHUNK)]

    row_ids = [lax.iota(jnp.int32, _L) + rc * _L for rc in range(_CH // _L)]

    st = []
    for c in range(_NCHUNK):
        gd[c].wait()
        xd[c].wait()
        rb = rows[c % _NBUF]
        xoff = c * _CH

        @plsc.parallel_loop(0, _D, unroll=2)
        def f_body(f, rb=rb, xoff=xoff):
            gcol = jnp.full((_L,), 0, jnp.int32) + f
            bcol = gcol + _D
            for rc in range(_CH // _L):
                g = plsc.load_gather(rb, [row_ids[rc], gcol])
                b = plsc.load_gather(rb, [row_ids[rc], bcol])
                xx = xt_v[f, pl.ds(xoff + rc * _L, _L)]
                xt_v[f, pl.ds(xoff + rc * _L, _L)] = g * xx + b

        st.append(pltpu.async_copy(
            xt_v.at[:, pl.ds(xoff, _CH)],
            out_hbm.at[:, pl.ds(base + xoff, _CH)], sem_st))
        if c + _NBUF < _NCHUNK:
            cn = c + _NBUF
            gd[cn] = pltpu.async_copy(
                film_hbm.at[idx_v.at[pl.ds(cn * _CH, _CH)]],
                rows[cn % _NBUF].at[:, pl.ds(0, 2 * _D)], sems[cn % _NBUF])

    for d in st:
        d.wait()


@jax.jit
def _film(xt, idx32, film):
    mesh = plsc.VectorSubcoreMesh(core_axis_name="c", subcore_axis_name="s")
    return pl.kernel(
        _film_body,
        out_type=jax.ShapeDtypeStruct((_D, _B), jnp.float32),
        mesh=mesh,
        compiler_params=pltpu.CompilerParams(needs_layout_passes=False),
        scratch_types=[
            pltpu.VMEM((_BPW,), jnp.int32),
            pltpu.VMEM((_D, _BPW), jnp.float32),
            pltpu.VMEM((_CH, _PITCH), jnp.float32),
            pltpu.VMEM((_CH, _PITCH), jnp.float32),
            pltpu.SemaphoreType.DMA,
            pltpu.SemaphoreType.DMA,
            pltpu.SemaphoreType.DMA,
            pltpu.SemaphoreType.DMA,
        ],
    )(xt, idx32, film)


def kernel(x, cell_line, film):
    idx32 = cell_line.astype(jnp.int32)
    out_t = _film(x.T, idx32, film)
    return (out_t.T, cell_line)


# speedup vs baseline: 1.4102x; 1.4102x over previous
import jax
import jax.numpy as jnp
from jax import lax
from jax.experimental import pallas as pl
from jax.experimental.pallas import tpu as pltpu
from jax.experimental.pallas import tpu_sc as plsc

_info = plsc.get_sparse_core_info()
_NC, _NS, _L = _info.num_cores, _info.num_subcores, _info.num_lanes
_NW = _NC * _NS

_B = 16384
_D = 64
_BPW = _B // _NW
_CH = 128
_NCHUNK = _BPW // _CH
_NBUF = 3


def _film_body(x_hbm, idx_hbm, film_hbm, out_hbm,
               idx_v, x_v, rows0, rows1, rows2,
               sem_x, sem_g0, sem_g1, sem_g2, sem_st):
    wid = lax.axis_index("s") * _NC + lax.axis_index("c")
    base = wid * _BPW

    pltpu.sync_copy(idx_hbm.at[pl.ds(base, _BPW)], idx_v)

    rows = (rows0, rows1, rows2)
    sems = (sem_g0, sem_g1, sem_g2)
    gd = [None] * _NCHUNK
    for c in range(_NBUF):
        gd[c] = pltpu.async_copy(
            film_hbm.at[idx_v.at[pl.ds(c * _CH, _CH)]], rows[c], sems[c])
    xd = [pltpu.async_copy(x_hbm.at[pl.ds(base + c * _CH, _CH)],
                           x_v.at[pl.ds(c * _CH, _CH)], sem_x)
          for c in range(_NCHUNK)]

    st = []
    for c in range(_NCHUNK):
        gd[c].wait()
        xd[c].wait()
        rb = rows[c % _NBUF]
        xoff = c * _CH

        @plsc.parallel_loop(0, _CH, unroll=4)
        def row_body(r, rb=rb, xoff=xoff):
            for j in range(_D // _L):
                g = rb[r, pl.ds(j * _L, _L)]
                b = rb[r, pl.ds(_D + j * _L, _L)]
                xx = x_v[xoff + r, pl.ds(j * _L, _L)]
                x_v[xoff + r, pl.ds(j * _L, _L)] = g * xx + b

        st.append(pltpu.async_copy(
            x_v.at[pl.ds(xoff, _CH)],
            out_hbm.at[pl.ds(base + xoff, _CH)], sem_st))
        if c + _NBUF < _NCHUNK:
            cn = c + _NBUF
            gd[cn] = pltpu.async_copy(
                film_hbm.at[idx_v.at[pl.ds(cn * _CH, _CH)]],
                rows[cn % _NBUF], sems[cn % _NBUF])

    for d in st:
        d.wait()


@jax.jit
def _film(x, idx32, film):
    mesh = plsc.VectorSubcoreMesh(core_axis_name="c", subcore_axis_name="s")
    return pl.kernel(
        _film_body,
        out_type=jax.ShapeDtypeStruct((_B, _D), jnp.float32),
        mesh=mesh,
        compiler_params=pltpu.CompilerParams(use_tc_tiling_on_sc=True),
        scratch_types=[
            pltpu.VMEM((_BPW,), jnp.int32),
            pltpu.VMEM((_BPW, _D), jnp.float32),
            pltpu.VMEM((_CH, 2 * _D), jnp.float32),
            pltpu.VMEM((_CH, 2 * _D), jnp.float32),
            pltpu.VMEM((_CH, 2 * _D), jnp.float32),
            pltpu.SemaphoreType.DMA,
            pltpu.SemaphoreType.DMA,
            pltpu.SemaphoreType.DMA,
            pltpu.SemaphoreType.DMA,
            pltpu.SemaphoreType.DMA,
        ],
    )(x, idx32, film)


def kernel(x, cell_line, film):
    idx32 = cell_line.astype(jnp.int32)
    out = _film(x, idx32, film)
    return (out, cell_line)
